# baseline (speedup 1.0000x reference)
import jax
import jax.numpy as jnp
import numpy as np
from jax.experimental import pallas as pl
from jax.experimental.pallas import tpu as pltpu

_N, _D, _H, _O = 10000, 128, 128, 128
_BR = 256
_NR = 40
_BC = 1024
_NC = 10
_EDGE_COLS = _N - (_NC - 1) * _BC
_BX = 2000
_NH = _N // _BX
_A0 = _NH
_B0 = _NH + _NR
_NPAD = _NR * _BR
_BF = 1024
_NF = 10
_NTRI = _NF * (_NF + 1) // 2
_S = _B0 + _NTRI

def _build_table() -> np.ndarray:
    tbl = np.zeros((4, _S), np.int32)
    for s in range(_S):
        tbl[0, s] = min(max(_NR - 1 - max(s - _A0, 0), 0), _NR - 1)
        tbl[3, s] = min(s, _NH - 1)
    t = 0
    for f in range(_NF):
        for kb in range(f + 1):
            tbl[1, _B0 + t] = f
            tbl[2, _B0 + t] = kb
            t += 1
    return tbl


_TBL = _build_table()


def _gcn_kernel(tbl_ref, x_ref, ga_ref, gb_ref, w1_ref, b1_ref, w2_ref,
                b2_ref, out_ref, hh_scr, h2_scr, oacc_scr, pub_sem):
    s = pl.program_id(0)

    @pl.when(s < _A0)
    def _phase_h1():
        rows = pl.ds(s * _BX, _BX)
        hh_scr[rows, :_H] = (
            jnp.dot(x_ref[...], w1_ref[...],
                    preferred_element_type=jnp.float32) + b1_ref[...])
        hh_scr[rows, _H:] = jnp.zeros((_BX, _O), jnp.float32)

    @pl.when((s >= _A0) & (s < _B0))
    def _phase_a():
        i = tbl_ref[0, s]

        @pl.when((i % 4 == 3) & (i < _NR - 1))
        def _pub_wait():
            pltpu.make_async_copy(
                h2_scr.at[pl.ds((i + 1) * _BR, _BC), :],
                hh_scr.at[pl.ds((i + 1) * _BR, _BC), _H:],
                pub_sem).wait()

        g = ga_ref[...]
        big = jnp.dot(g, hh_scr[pl.ds(0, _N), :],
                      preferred_element_type=jnp.float32)
        oacc_scr[pl.ds(i * _BR, _BR), :] = big[:, _H:]
        h2_scr[pl.ds(i * _BR, _BR), :] = (
            jnp.dot(jnp.maximum(big[:, :_H], 0.0), w2_ref[...],
                    preferred_element_type=jnp.float32) + b2_ref[...])

        @pl.when(s == _A0)
        def _zero_tail():
            h2_scr[pl.ds(_N, _NPAD - _N), :] = jnp.zeros(
                (_NPAD - _N, _O), jnp.float32)

        @pl.when((i % 4 == 0) & (i > 0))
        def _publish():
            pltpu.make_async_copy(
                h2_scr.at[pl.ds(i * _BR, _BC), :],
                hh_scr.at[pl.ds(i * _BR, _BC), _H:],
                pub_sem).start()

    @pl.when(s >= _B0)
    def _phase_b():
        f = tbl_ref[1, s]
        kb = tbl_ref[2, s]
        h2s = h2_scr[pl.ds(kb * _BC, _BC), :]
        row = pl.ds(f * _BF, _BF)

        @pl.when((kb < f) & (kb < _NC - 1))
        def _mid():
            oacc_scr[row, :] = oacc_scr[row, :] + jnp.dot(
                gb_ref[...], h2s, preferred_element_type=jnp.float32)

        @pl.when((kb == f) & (kb < _NC - 1))
        def _last():
            out_ref[...] = oacc_scr[row, :] + jnp.dot(
                gb_ref[...], h2s, preferred_element_type=jnp.float32)

        @pl.when(kb == _NC - 1)
        def _edge():
            lane = jax.lax.broadcasted_iota(jnp.int32, (_BF, _BC), 1)
            g = jnp.where(lane < _EDGE_COLS, gb_ref[...], 0.0)
            out_ref[...] = oacc_scr[row, :] + jnp.dot(
                g, h2s, preferred_element_type=jnp.float32)


def kernel(x, gcn, W1, b1, W2, b2):
    b1r = b1.reshape(1, _H)
    b2r = b2.reshape(1, _O)

    grid_spec = pltpu.PrefetchScalarGridSpec(
        num_scalar_prefetch=1,
        grid=(_S,),
        in_specs=[
            pl.BlockSpec((_BX, _D), lambda s, tbl: (tbl[3, s], 0)),
            pl.BlockSpec((_BR, _N), lambda s, tbl: (tbl[0, s], 0)),
            pl.BlockSpec((_BF, _BC), lambda s, tbl: (tbl[1, s], tbl[2, s])),
            pl.BlockSpec((_D, _H), lambda s, tbl: (0, 0)),
            pl.BlockSpec((1, _H), lambda s, tbl: (0, 0)),
            pl.BlockSpec((_H, _O), lambda s, tbl: (0, 0)),
            pl.BlockSpec((1, _O), lambda s, tbl: (0, 0)),
        ],
        out_specs=pl.BlockSpec((_BF, _O), lambda s, tbl: (tbl[1, s], 0)),
        scratch_shapes=[
            pltpu.VMEM((_NPAD, _H + _O), jnp.float32),
            pltpu.VMEM((_NPAD, _O), jnp.float32),
            pltpu.VMEM((_NPAD, _O), jnp.float32),
            pltpu.SemaphoreType.DMA,
        ],
    )

    out = pl.pallas_call(
        _gcn_kernel,
        grid_spec=grid_spec,
        out_shape=jax.ShapeDtypeStruct((_N, _O), jnp.float32),
        compiler_params=pltpu.CompilerParams(
            dimension_semantics=("arbitrary",)),
    )(jnp.asarray(_TBL), x, gcn, gcn, W1, b1r, W2, b2r)

    return out


# speedup vs baseline: 1.0060x; 1.0060x over previous
import jax
import jax.numpy as jnp
import numpy as np
from jax.experimental import pallas as pl
from jax.experimental.pallas import tpu as pltpu

_N, _D, _H, _O = 10000, 128, 128, 128
_BR = 256
_NR = 40
_BC = 1024
_NC = 10
_EDGE_COLS = _N - (_NC - 1) * _BC
_BX = 2000
_NH = _N // _BX
_A0 = _NH
_B0 = _NH + _NR
_NPAD = _NR * _BR
_BF = 1024
_NF = 10
_NTRI = _NF * (_NF + 1) // 2
_S = _B0 + _NTRI
_NFC = 2
_NTRIC = _NFC * (_NFC + 1) // 2

def _build_table() -> np.ndarray:
    tbl = np.zeros((6, _S), np.int32)
    for s in range(_S):
        tbl[0, s] = min(max(_NR - 1 - max(s - _A0, 0), 0), _NR - 1)
        tbl[3, s] = min(s, _NH - 1)
        tbl[4, s] = _NFC
        tbl[5, s] = 0
    t = 0
    for f in range(_NF):
        for kb in range(f + 1):
            tbl[1, _B0 + t] = f
            tbl[2, _B0 + t] = kb
            if f >= _NFC:
                tbl[4, _B0 + t] = f
                tbl[5, _B0 + t] = kb
            t += 1
    return tbl


_TBL = _build_table()


def _gcn_kernel(tbl_ref, x_ref, ga_ref, gb_ref, w1_ref, b1_ref, w2_ref,
                b2_ref, out_ref, hh_scr, h2_scr, oacc_scr, tri_scr, pub_sem):
    s = pl.program_id(0)

    @pl.when(s < _A0)
    def _phase_h1():
        rows = pl.ds(s * _BX, _BX)
        hh_scr[rows, :_H] = (
            jnp.dot(x_ref[...], w1_ref[...],
                    preferred_element_type=jnp.float32) + b1_ref[...])
        hh_scr[rows, _H:] = jnp.zeros((_BX, _O), jnp.float32)

    @pl.when((s >= _A0) & (s < _B0))
    def _phase_a():
        i = tbl_ref[0, s]

        @pl.when((i % 4 == 3) & (i < _NR - 1))
        def _pub_wait():
            pltpu.make_async_copy(
                h2_scr.at[pl.ds((i + 1) * _BR, _BC), :],
                hh_scr.at[pl.ds((i + 1) * _BR, _BC), _H:],
                pub_sem).wait()

        g = ga_ref[...]
        big = jnp.dot(g, hh_scr[pl.ds(0, _N), :],
                      preferred_element_type=jnp.float32)
        oacc_scr[pl.ds(i * _BR, _BR), :] = big[:, _H:]
        h2_scr[pl.ds(i * _BR, _BR), :] = (
            jnp.dot(jnp.maximum(big[:, :_H], 0.0), w2_ref[...],
                    preferred_element_type=jnp.float32) + b2_ref[...])

        @pl.when(s == _A0)
        def _zero_tail():
            h2_scr[pl.ds(_N, _NPAD - _N), :] = jnp.zeros(
                (_NPAD - _N, _O), jnp.float32)

        @pl.when((i % 4 == 0) & (i > 0))
        def _publish():
            pltpu.make_async_copy(
                h2_scr.at[pl.ds(i * _BR, _BC), :],
                hh_scr.at[pl.ds(i * _BR, _BC), _H:],
                pub_sem).start()

        @pl.when(i < _NFC * 4)
        def _cache_fill():
            fa = i // 4
            r = i % 4
            base = (fa * (fa + 1) // 2) * _BC + r * _BR
            for kb in range(_NFC):
                @pl.when(kb <= fa)
                def _copy(kb=kb, base=base):
                    tri_scr[pl.ds(base + kb * _BC, _BR), :] = (
                        ga_ref[:, kb * _BC:(kb + 1) * _BC])

    @pl.when(s >= _B0)
    def _phase_b():
        f = tbl_ref[1, s]
        kb = tbl_ref[2, s]
        h2s = h2_scr[pl.ds(kb * _BC, _BC), :]
        row = pl.ds(f * _BF, _BF)

        @pl.when(f < _NFC)
        def _cached():
            tri_off = (f * (f + 1) // 2 + kb) * _BC
            g = tri_scr[pl.ds(tri_off, _BF), :]
            acc = oacc_scr[row, :] + jnp.dot(
                g, h2s, preferred_element_type=jnp.float32)

            @pl.when(kb < f)
            def _mid_c():
                oacc_scr[row, :] = acc

            @pl.when(kb == f)
            def _last_c():
                out_ref[...] = acc

        @pl.when((f >= _NFC) & (kb < _NC - 1))
        def _hbm():
            acc = oacc_scr[row, :] + jnp.dot(
                gb_ref[...], h2s, preferred_element_type=jnp.float32)

            @pl.when(kb < f)
            def _mid_h():
                oacc_scr[row, :] = acc

            @pl.when(kb == f)
            def _last_h():
                out_ref[...] = acc

        @pl.when(kb == _NC - 1)
        def _edge():
            lane = jax.lax.broadcasted_iota(jnp.int32, (_BF, _BC), 1)
            g = jnp.where(lane < _EDGE_COLS, gb_ref[...], 0.0)
            out_ref[...] = oacc_scr[row, :] + jnp.dot(
                g, h2s, preferred_element_type=jnp.float32)


def kernel(x, gcn, W1, b1, W2, b2):
    b1r = b1.reshape(1, _H)
    b2r = b2.reshape(1, _O)

    grid_spec = pltpu.PrefetchScalarGridSpec(
        num_scalar_prefetch=1,
        grid=(_S,),
        in_specs=[
            pl.BlockSpec((_BX, _D), lambda s, tbl: (tbl[3, s], 0)),
            pl.BlockSpec((_BR, _N), lambda s, tbl: (tbl[0, s], 0)),
            pl.BlockSpec((_BF, _BC), lambda s, tbl: (tbl[4, s], tbl[5, s])),
            pl.BlockSpec((_D, _H), lambda s, tbl: (0, 0)),
            pl.BlockSpec((1, _H), lambda s, tbl: (0, 0)),
            pl.BlockSpec((_H, _O), lambda s, tbl: (0, 0)),
            pl.BlockSpec((1, _O), lambda s, tbl: (0, 0)),
        ],
        out_specs=pl.BlockSpec((_BF, _O), lambda s, tbl: (tbl[1, s], 0)),
        scratch_shapes=[
            pltpu.VMEM((_NPAD, _H + _O), jnp.float32),
            pltpu.VMEM((_NPAD, _O), jnp.float32),
            pltpu.VMEM((_NPAD, _O), jnp.float32),
            pltpu.VMEM((_NTRIC * _BC, _BC), jnp.float32),
            pltpu.SemaphoreType.DMA,
        ],
    )

    out = pl.pallas_call(
        _gcn_kernel,
        grid_spec=grid_spec,
        out_shape=jax.ShapeDtypeStruct((_N, _O), jnp.float32),
        compiler_params=pltpu.CompilerParams(
            dimension_semantics=("arbitrary",),
            vmem_limit_bytes=66_800_000),
    )(jnp.asarray(_TBL), x, gcn, gcn, W1, b1r, W2, b2r)

    return out
